# Initial kernel scaffold; baseline (speedup 1.0000x reference)
import jax
import jax.numpy as jnp
from jax.experimental import pallas as pl


def kernel(positions, boxVectors, species, embed_table, W_r, b_r, W_out):
    raise NotImplementedError("write your pallas kernel here")


# speedup vs baseline: 4.2603x; 4.2603x over previous
import functools
import math

import jax
import jax.numpy as jnp
from jax.experimental import pallas as pl
from jax.experimental.pallas import tpu as pltpu

N = 4096
D = 128
NB = 8
R_MAX = 5.0
CI = 8
NSTEPS = N // CI


def _dense_kernel(gm_ref, fcols_ref, frT_ref, hT_ref, wrT_ref, bcol_ref,
                  woT_ref, out_ref, acc_ref):
    step = pl.program_id(0)

    @pl.when(step == 0)
    def _init():
        acc_ref[...] = jnp.zeros_like(acc_ref)

    g00 = gm_ref[0, 0]
    g11 = gm_ref[0, 1]
    g22 = gm_ref[0, 2]
    g01 = gm_ref[0, 3]
    g02 = gm_ref[0, 4]
    g12 = gm_ref[0, 5]

    fi = fcols_ref[...]
    fj0 = frT_ref[0:1, :]
    fj1 = frT_ref[1:2, :]
    fj2 = frT_ref[2:3, :]
    d0 = fi[:, 0:1] - fj0
    d1 = fi[:, 1:2] - fj1
    d2 = fi[:, 2:3] - fj2
    d0 = d0 - jnp.round(d0)
    d1 = d1 - jnp.round(d1)
    d2 = d2 - jnp.round(d2)
    r2 = (g00 * d0 * d0 + g11 * d1 * d1 + g22 * d2 * d2
          + 2.0 * (g01 * d0 * d1 + g02 * d0 * d2 + g12 * d1 * d2))
    r2s = jnp.maximum(r2, 1e-24)
    inv_r = jax.lax.rsqrt(r2s)
    r = r2s * inv_r
    x = r * (1.0 / R_MAX)
    x2 = x * x
    x4 = x2 * x2
    x6 = x4 * x2
    env = 1.0 - x6 * (28.0 - x * (48.0 - 21.0 * x))
    pref = math.sqrt(2.0 / R_MAX)
    amp = jnp.where(x < 1.0, env, 0.0) * (pref * inv_r)
    theta = r * (math.pi / R_MAX)

    jota = jax.lax.broadcasted_iota(jnp.int32, (1, N), 1)
    nvec = jax.lax.broadcasted_iota(jnp.int32, (NB, 1), 0).astype(
        jnp.float32) + 1.0

    wrT = wrT_ref[...]
    bcol = bcol_ref[...]

    for s in range(CI):
        igl = step * CI + s
        th_s = theta[s:s + 1, :]
        rbT = jnp.sin(nvec * th_s) * amp[s:s + 1, :]
        z = jnp.dot(wrT, rbT, preferred_element_type=jnp.float32) + bcol
        sil = z * jax.nn.sigmoid(z)
        m = (r2[s:s + 1, :] < R_MAX * R_MAX) & (jota != igl)
        hm = jnp.where(m, hT_ref[0, :, s:s + 1], 0.0)
        acc_ref[...] += hm * sil

    @pl.when(step == NSTEPS - 1)
    def _epilogue():
        agg = acc_ref[...]
        silu_agg = agg * jax.nn.sigmoid(agg)
        e = jnp.dot(woT_ref[...], silu_agg, preferred_element_type=jnp.float32)
        out_ref[...] = jnp.sum(e, axis=1, keepdims=True)


@jax.jit
def kernel(positions, boxVectors, species, embed_table, W_r, b_r, W_out):
    pos = positions * 10.0
    box = boxVectors.astype(jnp.float32) * 10.0
    inv_box = jnp.linalg.inv(box)
    frac = pos @ inv_box
    G = box @ box.T
    gm = jnp.stack([G[0, 0], G[1, 1], G[2, 2],
                    G[0, 1], G[0, 2], G[1, 2],
                    jnp.float32(0), jnp.float32(0)]).reshape(1, 8)
    h = embed_table[species]
    hT3 = h.T.reshape(D, NSTEPS, CI).swapaxes(0, 1)
    fcols = jnp.pad(frac, ((0, 0), (0, 125)))
    frT = jnp.pad(frac.T, ((0, 5), (0, 0)))
    wrT = W_r.T
    bcol = b_r.reshape(D, 1)
    woT = W_out.T

    out = pl.pallas_call(
        _dense_kernel,
        grid=(NSTEPS,),
        in_specs=[
            pl.BlockSpec(memory_space=pltpu.SMEM),
            pl.BlockSpec((CI, 128), lambda i: (i, 0)),
            pl.BlockSpec((8, N), lambda i: (0, 0)),
            pl.BlockSpec((1, D, CI), lambda i: (i, 0, 0)),
            pl.BlockSpec((D, NB), lambda i: (0, 0)),
            pl.BlockSpec((D, 1), lambda i: (0, 0)),
            pl.BlockSpec((1, D), lambda i: (0, 0)),
        ],
        out_specs=pl.BlockSpec((1, 1), lambda i: (0, 0)),
        out_shape=jax.ShapeDtypeStruct((1, 1), jnp.float32),
        scratch_shapes=[pltpu.VMEM((D, N), jnp.float32)],
        compiler_params=pltpu.CompilerParams(
            dimension_semantics=("arbitrary",)),
    )(gm, fcols, frT, hT3, wrT, bcol, woT)
    return out.reshape(1)


# speedup vs baseline: 11.7625x; 2.7610x over previous
import functools
import math

import jax
import jax.numpy as jnp
from jax import lax
from jax.experimental import pallas as pl
from jax.experimental.pallas import tpu as pltpu
from jax.experimental.pallas import tpu_sc as plsc

N = 4096
D = 128
NB = 8
R_MAX = 5.0
CI = 8
NSTEPS_A = N // CI
K = 112
NSTEPS_D = K // CI
SENT = 1e6

NWORKERS = 32
ROWS_PER_W = N // NWORKERS
RB = 16


def _dist_kernel(gm_ref, fcols_ref, frT_ref, out_ref):
    g00 = gm_ref[0, 0]
    g11 = gm_ref[0, 1]
    g22 = gm_ref[0, 2]
    g01 = gm_ref[0, 3]
    g02 = gm_ref[0, 4]
    g12 = gm_ref[0, 5]
    fi = fcols_ref[...]
    d0 = fi[:, 0:1] - frT_ref[0:1, :]
    d1 = fi[:, 1:2] - frT_ref[1:2, :]
    d2 = fi[:, 2:3] - frT_ref[2:3, :]
    d0 = d0 - jnp.round(d0)
    d1 = d1 - jnp.round(d1)
    d2 = d2 - jnp.round(d2)
    r2 = (g00 * d0 * d0 + g11 * d1 * d1 + g22 * d2 * d2
          + 2.0 * (g01 * d0 * d1 + g02 * d0 * d2 + g12 * d1 * d2))
    jota = lax.broadcasted_iota(jnp.int32, (1, N), 1)
    rowi = (lax.broadcasted_iota(jnp.int32, (CI, 1), 0)
            + pl.program_id(0) * CI)
    out_ref[...] = jnp.where(jota == rowi, SENT, r2)


def _sc_compact(r2_hbm, spec_hbm, r2c_hbm, spc_hbm,
                rows_v, spec_v, cb_r2, cb_sp, t_r2, t_sp):
    wid = lax.axis_index("s") * 2 + lax.axis_index("c")
    j0 = wid * ROWS_PER_W
    pltpu.sync_copy(spec_hbm, spec_v)
    def do_row(col, _):
        lanes = lax.iota(jnp.int32, 16)
        sentv = jnp.full((16,), SENT, jnp.float32)
        j = j0 + col
        pltpu.sync_copy(r2_hbm.at[j], rows_v)
        ptrv = jnp.zeros((16,), jnp.int32)
        c25 = jnp.full((16,), R_MAX * R_MAX, jnp.float32)
        c256 = jnp.full((16,), 256, jnp.int32)
        c255 = jnp.full((16,), 255, jnp.int32)
        kv = jnp.full((16,), K, jnp.int32)
        l15 = jnp.full((16,), 15, jnp.int32)
        for t in range(9):
            cb_r2[pl.ds(16 * t, 16)] = sentv
            cb_sp[pl.ds(16 * t, 16)] = jnp.zeros((16,), jnp.int32)
        for v in range(N // 16):
            vals = rows_v[pl.ds(16 * v, 16)]
            hit = jnp.maximum(jnp.sign(c25 - vals), 0.0).astype(jnp.int32)
            pos = plsc.cumsum(hit)
            pv = ptrv + pos
            idx = (pv - c256) * hit + c255
            plsc.store_scatter(cb_r2, [idx], vals)
            spv = spec_v[pl.ds(16 * v, 16)]
            plsc.store_scatter(cb_sp, [idx], spv)
            bc = lax.gather(
                pv, l15[:, None],
                lax.GatherDimensionNumbers(offset_dims=(),
                                           collapsed_slice_dims=(0,),
                                           start_index_map=(0,)),
                (1,), mode=lax.GatherScatterMode.PROMISE_IN_BOUNDS)
            ptrv = jnp.minimum(bc, kv)
        colv = jnp.full((16,), col, jnp.int32)
        for t in range(K // 16):
            flat = (lanes + jnp.full((16,), 16 * t, jnp.int32)) * ROWS_PER_W \
                + colv
            plsc.store_scatter(t_r2, [flat], cb_r2[pl.ds(16 * t, 16)])
            plsc.store_scatter(t_sp, [flat], cb_sp[pl.ds(16 * t, 16)])
        return 0

    lax.fori_loop(0, ROWS_PER_W, do_row, 0)
    pltpu.sync_copy(t_r2, r2c_hbm.at[wid])
    pltpu.sync_copy(t_sp, spc_hbm.at[wid])


@functools.cache
def _get_compact_call():
    return pl.kernel(
        _sc_compact,
        out_type=(
            jax.ShapeDtypeStruct((NWORKERS, K * ROWS_PER_W), jnp.float32),
            jax.ShapeDtypeStruct((NWORKERS, K * ROWS_PER_W), jnp.int32)),
        mesh=plsc.VectorSubcoreMesh(core_axis_name="c", subcore_axis_name="s"),
        compiler_params=pltpu.CompilerParams(needs_layout_passes=False),
        scratch_types=[
            pltpu.VMEM((N,), jnp.float32),
            pltpu.VMEM((N,), jnp.int32),
            pltpu.VMEM((256,), jnp.float32),
            pltpu.VMEM((256,), jnp.int32),
            pltpu.VMEM((K * ROWS_PER_W,), jnp.float32),
            pltpu.VMEM((K * ROWS_PER_W,), jnp.int32),
        ],
    )


def _msg_kernel(r2c_ref, spc_ref, wrT_ref, bcol_ref, woT_ref, ett_ref,
                out_ref, acc_ref):
    step = pl.program_id(0)

    @pl.when(step == 0)
    def _init():
        acc_ref[...] = jnp.zeros_like(acc_ref)

    r2 = r2c_ref[...]
    r2s = jnp.maximum(r2, 1e-24)
    inv_r = lax.rsqrt(r2s)
    r = r2s * inv_r
    x = r * (1.0 / R_MAX)
    x2 = x * x
    x4 = x2 * x2
    x6 = x4 * x2
    env = 1.0 - x6 * (28.0 - x * (48.0 - 21.0 * x))
    pref = math.sqrt(2.0 / R_MAX)
    amp = jnp.where(x < 1.0, env, 0.0) * (pref * inv_r)
    theta = r * (math.pi / R_MAX)
    spc = spc_ref[...]

    nvec = lax.broadcasted_iota(jnp.int32, (NB, 1), 0).astype(jnp.float32) + 1.0
    iota16 = lax.broadcasted_iota(jnp.int32, (16, 1), 0)
    wrT = wrT_ref[...]
    bcol = bcol_ref[...]
    ett = ett_ref[...]

    for s in range(CI):
        m = r2[s:s + 1, :] < R_MAX * R_MAX
        rbT = jnp.sin(nvec * theta[s:s + 1, :]) * amp[s:s + 1, :]
        z = jnp.dot(wrT, rbT, preferred_element_type=jnp.float32) + bcol
        sil = z * jax.nn.sigmoid(z)
        oh = (spc[s:s + 1, :] == iota16).astype(jnp.float32)
        hT_s = jnp.dot(ett, oh, preferred_element_type=jnp.float32)
        acc_ref[...] += jnp.where(m, hT_s, 0.0) * sil

    @pl.when(step == NSTEPS_D - 1)
    def _epilogue():
        agg = acc_ref[...]
        silu_agg = agg * jax.nn.sigmoid(agg)
        e = jnp.dot(woT_ref[...], silu_agg, preferred_element_type=jnp.float32)
        out_ref[...] = jnp.sum(e, axis=1, keepdims=True)


@jax.jit
def kernel(positions, boxVectors, species, embed_table, W_r, b_r, W_out):
    pos = positions * 10.0
    box = boxVectors.astype(jnp.float32) * 10.0
    inv_box = jnp.linalg.inv(box)
    frac = pos @ inv_box
    G = box @ box.T
    gm = jnp.stack([G[0, 0], G[1, 1], G[2, 2],
                    G[0, 1], G[0, 2], G[1, 2],
                    jnp.float32(0), jnp.float32(0)]).reshape(1, 8)
    fcols = jnp.pad(frac, ((0, 0), (0, 125)))
    frT = jnp.pad(frac.T, ((0, 5), (0, 0)))
    spec32 = species.astype(jnp.int32)
    wrT = W_r.T
    bcol = b_r.reshape(D, 1)
    woT = W_out.T
    ett = jnp.pad(embed_table.T, ((0, 0), (0, 6)))

    r2d = pl.pallas_call(
        _dist_kernel,
        grid=(NSTEPS_A,),
        in_specs=[
            pl.BlockSpec(memory_space=pltpu.SMEM),
            pl.BlockSpec((CI, 128), lambda i: (i, 0)),
            pl.BlockSpec((8, N), lambda i: (0, 0)),
        ],
        out_specs=pl.BlockSpec((CI, N), lambda i: (i, 0)),
        out_shape=jax.ShapeDtypeStruct((N, N), jnp.float32),
        compiler_params=pltpu.CompilerParams(
            dimension_semantics=("arbitrary",)),
    )(gm, fcols, frT)

    r2c_w, spc_w = _get_compact_call()(r2d, spec32)
    r2c = r2c_w.reshape(NWORKERS, K, ROWS_PER_W).transpose(1, 0, 2).reshape(K, N)
    spc = spc_w.reshape(NWORKERS, K, ROWS_PER_W).transpose(1, 0, 2).reshape(K, N)

    out = pl.pallas_call(
        _msg_kernel,
        grid=(NSTEPS_D,),
        in_specs=[
            pl.BlockSpec((CI, N), lambda i: (i, 0)),
            pl.BlockSpec((CI, N), lambda i: (i, 0)),
            pl.BlockSpec((D, NB), lambda i: (0, 0)),
            pl.BlockSpec((D, 1), lambda i: (0, 0)),
            pl.BlockSpec((1, D), lambda i: (0, 0)),
            pl.BlockSpec((D, 16), lambda i: (0, 0)),
        ],
        out_specs=pl.BlockSpec((1, 1), lambda i: (0, 0)),
        out_shape=jax.ShapeDtypeStruct((1, 1), jnp.float32),
        scratch_shapes=[pltpu.VMEM((D, N), jnp.float32)],
        compiler_params=pltpu.CompilerParams(
            dimension_semantics=("arbitrary",)),
    )(r2c, spc, wrT, bcol, woT, ett)
    return out.reshape(1)


# speedup vs baseline: 21.0027x; 1.7856x over previous
import functools
import math

import jax
import jax.numpy as jnp
from jax import lax
from jax.experimental import pallas as pl
from jax.experimental.pallas import tpu as pltpu
from jax.experimental.pallas import tpu_sc as plsc

N = 4096
D = 128
NB = 8
R_MAX = 5.0
CI = 8
NSTEPS_A = N // CI
K = 112
NSTEPS_D = K // CI
SENT = 1e6

NWORKERS = 32
ROWS_PER_W = N // NWORKERS
RB = 16


def _dist_kernel(gm_ref, fcols_ref, frT_ref, ex_ref, out_ref, cnt_ref):
    g00 = gm_ref[0, 0]
    g11 = gm_ref[0, 1]
    g22 = gm_ref[0, 2]
    g01 = gm_ref[0, 3]
    g02 = gm_ref[0, 4]
    g12 = gm_ref[0, 5]
    fi = fcols_ref[...]
    d0 = fi[:, 0:1] - frT_ref[0:1, :]
    d1 = fi[:, 1:2] - frT_ref[1:2, :]
    d2 = fi[:, 2:3] - frT_ref[2:3, :]
    d0 = d0 - jnp.round(d0)
    d1 = d1 - jnp.round(d1)
    d2 = d2 - jnp.round(d2)
    r2 = (g00 * d0 * d0 + g11 * d1 * d1 + g22 * d2 * d2
          + 2.0 * (g01 * d0 * d1 + g02 * d0 * d2 + g12 * d1 * d2))
    jota = lax.broadcasted_iota(jnp.int32, (1, N), 1)
    rowi = (lax.broadcasted_iota(jnp.int32, (CI, 1), 0)
            + pl.program_id(0) * CI)
    r2p = jnp.where(jota == rowi, SENT, r2)
    out_ref[...] = r2p
    mask01 = jnp.where(r2p < R_MAX * R_MAX, 1.0, 0.0)
    cnt_ref[...] = jnp.dot(mask01, ex_ref[...],
                           preferred_element_type=jnp.float32
                           ).astype(jnp.int32)


def _sc_compact(r2_hbm, cnt_hbm, spec_hbm, r2c_hbm, spc_hbm,
                rows_v, cnt_w, spec_v, cb_r2, cb_sp, clist, blist, sbuf,
                t_r2, t_sp):
    wid = lax.axis_index("s") * 2 + lax.axis_index("c")
    j0 = wid * ROWS_PER_W
    pltpu.sync_copy(spec_hbm, spec_v)
    pltpu.sync_copy(cnt_hbm.at[wid], cnt_w)

    def bc15(x):
        l15 = jnp.full((16,), 15, jnp.int32)
        return lax.gather(
            x, l15[:, None],
            lax.GatherDimensionNumbers(offset_dims=(),
                                       collapsed_slice_dims=(0,),
                                       start_index_map=(0,)),
            (1,), mode=lax.GatherScatterMode.PROMISE_IN_BOUNDS)

    def do_row(col, _):
        lanes = lax.iota(jnp.int32, 16)
        sentv = jnp.full((16,), SENT, jnp.float32)
        onev = jnp.full((16,), 1, jnp.int32)
        kv = jnp.full((16,), K, jnp.int32)
        c25 = jnp.full((16,), R_MAX * R_MAX, jnp.float32)
        c256 = jnp.full((16,), 256, jnp.int32)
        c255 = jnp.full((16,), 255, jnp.int32)
        c512 = jnp.full((16,), 512, jnp.int32)
        c511 = jnp.full((16,), 511, jnp.int32)
        j = j0 + col
        pltpu.sync_copy(r2_hbm.at[j], rows_v)
        for t in range(9):
            cb_r2[pl.ds(16 * t, 16)] = sentv
            cb_sp[pl.ds(16 * t, 16)] = jnp.zeros((16,), jnp.int32)
        carry = jnp.zeros((16,), jnp.int32)
        nzc = jnp.zeros((16,), jnp.int32)
        for t in range(16):
            cv = cnt_w[pl.ds(col * 256 + 16 * t, 16)]
            ps = plsc.cumsum(cv)
            basev = carry + ps - cv
            carry = bc15(carry + ps)
            nz = jnp.minimum(cv, onev)
            nps = plsc.cumsum(nz)
            lpos = nzc + nps
            idx2 = (lpos - c512) * nz + c511
            cidv = lanes + jnp.full((16,), 16 * t, jnp.int32)
            plsc.store_scatter(clist, [idx2], cidv)
            plsc.store_scatter(blist, [idx2], jnp.minimum(basev, kv))
            nzc = bc15(nzc + nps)
        nnz = nzc[0]
        def do_chunk(k, _):
            c = clist[pl.ds(k, 16)][0]
            b = blist[pl.ds(k, 16)][0]
            vals = rows_v[pl.ds(16 * c, 16)]
            hit = jnp.maximum(jnp.sign(c25 - vals), 0.0).astype(jnp.int32)
            pos = plsc.cumsum(hit)
            bv = jnp.full((16,), b, jnp.int32)
            idx = (bv + pos - c256) * hit + c255
            plsc.store_scatter(cb_r2, [idx], vals)
            spv = spec_v[pl.ds(16 * c, 16)]
            plsc.store_scatter(cb_sp, [idx], spv)
            return 0

        lax.fori_loop(0, nnz, do_chunk, 0)
        colv = jnp.full((16,), col, jnp.int32)
        for t in range(K // 16):
            flat = (lanes + jnp.full((16,), 16 * t, jnp.int32)) * ROWS_PER_W \
                + colv
            plsc.store_scatter(t_r2, [flat], cb_r2[pl.ds(16 * t, 16)])
            plsc.store_scatter(t_sp, [flat], cb_sp[pl.ds(16 * t, 16)])
        return 0

    lax.fori_loop(0, ROWS_PER_W, do_row, 0)
    pltpu.sync_copy(t_r2, r2c_hbm.at[wid])
    pltpu.sync_copy(t_sp, spc_hbm.at[wid])


@functools.cache
def _get_compact_call():
    return pl.kernel(
        _sc_compact,
        out_type=(
            jax.ShapeDtypeStruct((NWORKERS, K * ROWS_PER_W), jnp.float32),
            jax.ShapeDtypeStruct((NWORKERS, K * ROWS_PER_W), jnp.int32)),
        mesh=plsc.VectorSubcoreMesh(core_axis_name="c", subcore_axis_name="s"),
        compiler_params=pltpu.CompilerParams(needs_layout_passes=False),
        scratch_types=[
            pltpu.VMEM((N,), jnp.float32),
            pltpu.VMEM((ROWS_PER_W * 256,), jnp.int32),
            pltpu.VMEM((N,), jnp.int32),
            pltpu.VMEM((256,), jnp.float32),
            pltpu.VMEM((256,), jnp.int32),
            pltpu.VMEM((512,), jnp.int32),
            pltpu.VMEM((512,), jnp.int32),
            pltpu.VMEM((16,), jnp.int32),
            pltpu.VMEM((K * ROWS_PER_W,), jnp.float32),
            pltpu.VMEM((K * ROWS_PER_W,), jnp.int32),
        ],
    )


def _msg_kernel(r2c_ref, spc_ref, wrT_ref, bcol_ref, woT_ref, ett_ref,
                out_ref, acc_ref):
    step = pl.program_id(0)

    @pl.when(step == 0)
    def _init():
        acc_ref[...] = jnp.zeros_like(acc_ref)

    r2 = r2c_ref[...]
    r2s = jnp.maximum(r2, 1e-24)
    inv_r = lax.rsqrt(r2s)
    r = r2s * inv_r
    x = r * (1.0 / R_MAX)
    x2 = x * x
    x4 = x2 * x2
    x6 = x4 * x2
    env = 1.0 - x6 * (28.0 - x * (48.0 - 21.0 * x))
    pref = math.sqrt(2.0 / R_MAX)
    amp = jnp.where(x < 1.0, env, 0.0) * (pref * inv_r)
    theta = r * (math.pi / R_MAX)
    spc = spc_ref[...]

    nvec = lax.broadcasted_iota(jnp.int32, (NB, 1), 0).astype(jnp.float32) + 1.0
    iota16 = lax.broadcasted_iota(jnp.int32, (16, 1), 0)
    wrT = wrT_ref[...]
    bcol = bcol_ref[...]
    ett = ett_ref[...]

    for s in range(CI):
        m = r2[s:s + 1, :] < R_MAX * R_MAX
        rbT = jnp.sin(nvec * theta[s:s + 1, :]) * amp[s:s + 1, :]
        z = jnp.dot(wrT, rbT, preferred_element_type=jnp.float32,
                    precision=lax.Precision.HIGHEST) + bcol
        sil = z * jax.nn.sigmoid(z)
        oh = (spc[s:s + 1, :] == iota16).astype(jnp.float32)
        hT_s = jnp.dot(ett, oh, preferred_element_type=jnp.float32,
                       precision=lax.Precision.HIGHEST)
        acc_ref[...] += jnp.where(m, hT_s, 0.0) * sil

    @pl.when(step == NSTEPS_D - 1)
    def _epilogue():
        agg = acc_ref[...]
        silu_agg = agg * jax.nn.sigmoid(agg)
        e = jnp.dot(woT_ref[...], silu_agg, preferred_element_type=jnp.float32,
                    precision=lax.Precision.HIGHEST)
        out_ref[...] = jnp.sum(e, axis=1, keepdims=True)


@jax.jit
def kernel(positions, boxVectors, species, embed_table, W_r, b_r, W_out):
    pos = positions * 10.0
    box = boxVectors.astype(jnp.float32) * 10.0
    inv_box = jnp.linalg.inv(box)
    frac = pos @ inv_box
    G = box @ box.T
    gm = jnp.stack([G[0, 0], G[1, 1], G[2, 2],
                    G[0, 1], G[0, 2], G[1, 2],
                    jnp.float32(0), jnp.float32(0)]).reshape(1, 8)
    fcols = jnp.pad(frac, ((0, 0), (0, 125)))
    frT = jnp.pad(frac.T, ((0, 5), (0, 0)))
    spec32 = species.astype(jnp.int32)
    wrT = W_r.T
    bcol = b_r.reshape(D, 1)
    woT = W_out.T
    ett = jnp.pad(embed_table.T, ((0, 0), (0, 6)))
    iota_n = jnp.arange(N, dtype=jnp.int32)
    iota_c = jnp.arange(N // 16, dtype=jnp.int32)
    expander = (iota_n[:, None] // 16 == iota_c[None, :]).astype(jnp.float32)

    r2d, cnts = pl.pallas_call(
        _dist_kernel,
        grid=(NSTEPS_A,),
        in_specs=[
            pl.BlockSpec(memory_space=pltpu.SMEM),
            pl.BlockSpec((CI, 128), lambda i: (i, 0)),
            pl.BlockSpec((8, N), lambda i: (0, 0)),
            pl.BlockSpec((N, N // 16), lambda i: (0, 0)),
        ],
        out_specs=(pl.BlockSpec((CI, N), lambda i: (i, 0)),
                   pl.BlockSpec((CI, N // 16), lambda i: (i, 0))),
        out_shape=(jax.ShapeDtypeStruct((N, N), jnp.float32),
                   jax.ShapeDtypeStruct((N, N // 16), jnp.int32)),
        compiler_params=pltpu.CompilerParams(
            dimension_semantics=("arbitrary",)),
    )(gm, fcols, frT, expander)

    cnts_w = cnts.reshape(NWORKERS, ROWS_PER_W * (N // 16))
    r2c_w, spc_w = _get_compact_call()(r2d, cnts_w, spec32)
    r2c = r2c_w.reshape(NWORKERS, K, ROWS_PER_W).transpose(1, 0, 2).reshape(K, N)
    spc = spc_w.reshape(NWORKERS, K, ROWS_PER_W).transpose(1, 0, 2).reshape(K, N)

    out = pl.pallas_call(
        _msg_kernel,
        grid=(NSTEPS_D,),
        in_specs=[
            pl.BlockSpec((CI, N), lambda i: (i, 0)),
            pl.BlockSpec((CI, N), lambda i: (i, 0)),
            pl.BlockSpec((D, NB), lambda i: (0, 0)),
            pl.BlockSpec((D, 1), lambda i: (0, 0)),
            pl.BlockSpec((1, D), lambda i: (0, 0)),
            pl.BlockSpec((D, 16), lambda i: (0, 0)),
        ],
        out_specs=pl.BlockSpec((1, 1), lambda i: (0, 0)),
        out_shape=jax.ShapeDtypeStruct((1, 1), jnp.float32),
        scratch_shapes=[pltpu.VMEM((D, N), jnp.float32)],
        compiler_params=pltpu.CompilerParams(
            dimension_semantics=("arbitrary",)),
    )(r2c, spc, wrT, bcol, woT, ett)
    return out.reshape(1)


# speedup vs baseline: 24.0318x; 1.1442x over previous
import functools
import math

import jax
import jax.numpy as jnp
from jax import lax
from jax.experimental import pallas as pl
from jax.experimental.pallas import tpu as pltpu
from jax.experimental.pallas import tpu_sc as plsc

N = 4096
D = 128
NB = 8
R_MAX = 5.0
CI = 8
NSTEPS_A = N // CI
K = 112
NSTEPS_D = K // CI
SENT = 1e6

NWORKERS = 32
ROWS_PER_W = N // NWORKERS
RB = 16


def _dist_kernel(gm_ref, fcols_ref, frT_ref, ex_ref, out_ref, cnt_ref):
    fi = fcols_ref[...]
    d0 = fi[:, 0:1] - frT_ref[0:1, :]
    d1 = fi[:, 1:2] - frT_ref[1:2, :]
    d2 = fi[:, 2:3] - frT_ref[2:3, :]
    d0 = d0 - jnp.round(d0)
    d1 = d1 - jnp.round(d1)
    d2 = d2 - jnp.round(d2)
    db0 = d0.astype(jnp.bfloat16).astype(jnp.float32)
    db1 = d1.astype(jnp.bfloat16).astype(jnp.float32)
    db2 = d2.astype(jnp.bfloat16).astype(jnp.float32)
    v0 = db0 * gm_ref[0, 0] + db1 * gm_ref[0, 3] + db2 * gm_ref[0, 6]
    v1 = db0 * gm_ref[0, 1] + db1 * gm_ref[0, 4] + db2 * gm_ref[0, 7]
    v2 = db0 * gm_ref[0, 2] + db1 * gm_ref[0, 5] + db2 * gm_ref[0, 8]
    r2 = v0 * v0 + v1 * v1 + v2 * v2
    jota = lax.broadcasted_iota(jnp.int32, (1, N), 1)
    rowi = (lax.broadcasted_iota(jnp.int32, (CI, 1), 0)
            + pl.program_id(0) * CI)
    r2p = jnp.where(jota == rowi, SENT, r2)
    out_ref[...] = r2p
    mask01 = jnp.where(r2p < R_MAX * R_MAX, 1.0, 0.0)
    cnt_ref[...] = jnp.dot(mask01, ex_ref[...],
                           preferred_element_type=jnp.float32
                           ).astype(jnp.int32)


def _sc_compact(r2_hbm, cnt_hbm, spec_hbm, r2c_hbm, spc_hbm,
                rows_v, cnt_w, spec_v, cb_r2, cb_sp, clist, blist, sbuf,
                t_r2, t_sp):
    wid = lax.axis_index("s") * 2 + lax.axis_index("c")
    j0 = wid * ROWS_PER_W
    pltpu.sync_copy(spec_hbm, spec_v)
    pltpu.sync_copy(cnt_hbm.at[wid], cnt_w)

    def bc15(x):
        l15 = jnp.full((16,), 15, jnp.int32)
        return lax.gather(
            x, l15[:, None],
            lax.GatherDimensionNumbers(offset_dims=(),
                                       collapsed_slice_dims=(0,),
                                       start_index_map=(0,)),
            (1,), mode=lax.GatherScatterMode.PROMISE_IN_BOUNDS)

    def do_row(col, _):
        lanes = lax.iota(jnp.int32, 16)
        sentv = jnp.full((16,), SENT, jnp.float32)
        onev = jnp.full((16,), 1, jnp.int32)
        kv = jnp.full((16,), K, jnp.int32)
        c25 = jnp.full((16,), R_MAX * R_MAX, jnp.float32)
        c256 = jnp.full((16,), 256, jnp.int32)
        c255 = jnp.full((16,), 255, jnp.int32)
        c512 = jnp.full((16,), 512, jnp.int32)
        c511 = jnp.full((16,), 511, jnp.int32)
        j = j0 + col
        pltpu.sync_copy(r2_hbm.at[j], rows_v)
        for t in range(9):
            cb_r2[pl.ds(16 * t, 16)] = sentv
            cb_sp[pl.ds(16 * t, 16)] = jnp.zeros((16,), jnp.int32)
        carry = jnp.zeros((16,), jnp.int32)
        nzc = jnp.zeros((16,), jnp.int32)
        for t in range(16):
            cv = cnt_w[pl.ds(col * 256 + 16 * t, 16)]
            ps = plsc.cumsum(cv)
            basev = carry + ps - cv
            carry = bc15(carry + ps)
            nz = jnp.minimum(cv, onev)
            nps = plsc.cumsum(nz)
            lpos = nzc + nps
            idx2 = (lpos - c512) * nz + c511
            cidv = lanes + jnp.full((16,), 16 * t, jnp.int32)
            plsc.store_scatter(clist, [idx2], cidv)
            plsc.store_scatter(blist, [idx2], jnp.minimum(basev, kv))
            nzc = bc15(nzc + nps)
        nnz = nzc[0]
        def do_chunk(k, _):
            c = clist[pl.ds(k, 16)][0]
            b = blist[pl.ds(k, 16)][0]
            vals = rows_v[pl.ds(16 * c, 16)]
            hit = jnp.maximum(jnp.sign(c25 - vals), 0.0).astype(jnp.int32)
            pos = plsc.cumsum(hit)
            bv = jnp.full((16,), b, jnp.int32)
            idx = (bv + pos - c256) * hit + c255
            plsc.store_scatter(cb_r2, [idx], vals)
            spv = spec_v[pl.ds(16 * c, 16)]
            plsc.store_scatter(cb_sp, [idx], spv)
            return 0

        lax.fori_loop(0, nnz, do_chunk, 0)
        colv = jnp.full((16,), col, jnp.int32)
        for t in range(K // 16):
            flat = (lanes + jnp.full((16,), 16 * t, jnp.int32)) * ROWS_PER_W \
                + colv
            plsc.store_scatter(t_r2, [flat], cb_r2[pl.ds(16 * t, 16)])
            plsc.store_scatter(t_sp, [flat], cb_sp[pl.ds(16 * t, 16)])
        return 0

    lax.fori_loop(0, ROWS_PER_W, do_row, 0)
    pltpu.sync_copy(t_r2, r2c_hbm.at[wid])
    pltpu.sync_copy(t_sp, spc_hbm.at[wid])


@functools.cache
def _get_compact_call():
    return pl.kernel(
        _sc_compact,
        out_type=(
            jax.ShapeDtypeStruct((NWORKERS, K * ROWS_PER_W), jnp.float32),
            jax.ShapeDtypeStruct((NWORKERS, K * ROWS_PER_W), jnp.int32)),
        mesh=plsc.VectorSubcoreMesh(core_axis_name="c", subcore_axis_name="s"),
        compiler_params=pltpu.CompilerParams(needs_layout_passes=False),
        scratch_types=[
            pltpu.VMEM((N,), jnp.float32),
            pltpu.VMEM((ROWS_PER_W * 256,), jnp.int32),
            pltpu.VMEM((N,), jnp.int32),
            pltpu.VMEM((256,), jnp.float32),
            pltpu.VMEM((256,), jnp.int32),
            pltpu.VMEM((512,), jnp.int32),
            pltpu.VMEM((512,), jnp.int32),
            pltpu.VMEM((16,), jnp.int32),
            pltpu.VMEM((K * ROWS_PER_W,), jnp.float32),
            pltpu.VMEM((K * ROWS_PER_W,), jnp.int32),
        ],
    )


def _msg_kernel(r2c_ref, spc_ref, wrT_ref, bcol_ref, woT_ref, ett_ref,
                out_ref, acc_ref):
    step = pl.program_id(0)

    @pl.when(step == 0)
    def _init():
        acc_ref[...] = jnp.zeros_like(acc_ref)

    r2 = r2c_ref[...]
    r2s = jnp.maximum(r2, 1e-24)
    inv_r = lax.rsqrt(r2s)
    inv_r = inv_r * (1.5 - 0.5 * r2s * inv_r * inv_r)
    r = r2s * inv_r
    x = r * (1.0 / R_MAX)
    x2 = x * x
    x4 = x2 * x2
    x6 = x4 * x2
    env = 1.0 - x6 * (28.0 - x * (48.0 - 21.0 * x))
    pref = math.sqrt(2.0 / R_MAX)
    amp = jnp.where(x < 1.0, env, 0.0) * (pref * inv_r)
    theta = r * (math.pi / R_MAX)
    spc = spc_ref[...]

    nvec = lax.broadcasted_iota(jnp.int32, (NB, 1), 0).astype(jnp.float32) + 1.0
    iota16 = lax.broadcasted_iota(jnp.int32, (16, 1), 0)
    wrT = wrT_ref[...]
    bcol = bcol_ref[...]
    ett = ett_ref[...]

    for s in range(CI):
        m = r2[s:s + 1, :] < R_MAX * R_MAX
        rbT = jnp.sin(nvec * theta[s:s + 1, :]) * amp[s:s + 1, :]
        z = jnp.dot(wrT, rbT, preferred_element_type=jnp.float32) + bcol
        sil = z * jax.nn.sigmoid(z)
        oh = (spc[s:s + 1, :] == iota16).astype(jnp.float32)
        hT_s = jnp.dot(ett, oh, preferred_element_type=jnp.float32,
                       precision=lax.Precision.HIGHEST)
        acc_ref[...] += jnp.where(m, hT_s, 0.0) * sil

    @pl.when(step == NSTEPS_D - 1)
    def _epilogue():
        agg = acc_ref[...]
        silu_agg = agg * jax.nn.sigmoid(agg)
        e = jnp.dot(woT_ref[...], silu_agg, preferred_element_type=jnp.float32)
        out_ref[...] = jnp.sum(e, axis=1, keepdims=True)


@jax.jit
def kernel(positions, boxVectors, species, embed_table, W_r, b_r, W_out):
    pos = positions * 10.0
    box = boxVectors.astype(jnp.float32) * 10.0
    inv_box = jnp.linalg.inv(box)
    frac = pos @ inv_box
    boxb = box.astype(jnp.bfloat16).astype(jnp.float32)
    gm = jnp.pad(boxb.reshape(1, 9), ((0, 0), (0, 7)))
    fcols = jnp.pad(frac, ((0, 0), (0, 125)))
    frT = jnp.pad(frac.T, ((0, 5), (0, 0)))
    spec32 = species.astype(jnp.int32)
    wrT = W_r.T
    bcol = b_r.reshape(D, 1)
    woT = W_out.T
    ett = jnp.pad(embed_table.T, ((0, 0), (0, 6)))
    iota_n = jnp.arange(N, dtype=jnp.int32)
    iota_c = jnp.arange(N // 16, dtype=jnp.int32)
    expander = (iota_n[:, None] // 16 == iota_c[None, :]).astype(jnp.float32)

    r2d, cnts = pl.pallas_call(
        _dist_kernel,
        grid=(NSTEPS_A,),
        in_specs=[
            pl.BlockSpec(memory_space=pltpu.SMEM),
            pl.BlockSpec((CI, 128), lambda i: (i, 0)),
            pl.BlockSpec((8, N), lambda i: (0, 0)),
            pl.BlockSpec((N, N // 16), lambda i: (0, 0)),
        ],
        out_specs=(pl.BlockSpec((CI, N), lambda i: (i, 0)),
                   pl.BlockSpec((CI, N // 16), lambda i: (i, 0))),
        out_shape=(jax.ShapeDtypeStruct((N, N), jnp.float32),
                   jax.ShapeDtypeStruct((N, N // 16), jnp.int32)),
        compiler_params=pltpu.CompilerParams(
            dimension_semantics=("arbitrary",)),
    )(gm, fcols, frT, expander)

    cnts_w = cnts.reshape(NWORKERS, ROWS_PER_W * (N // 16))
    r2c_w, spc_w = _get_compact_call()(r2d, cnts_w, spec32)
    r2c = r2c_w.reshape(NWORKERS, K, ROWS_PER_W).transpose(1, 0, 2).reshape(K, N)
    spc = spc_w.reshape(NWORKERS, K, ROWS_PER_W).transpose(1, 0, 2).reshape(K, N)

    out = pl.pallas_call(
        _msg_kernel,
        grid=(NSTEPS_D,),
        in_specs=[
            pl.BlockSpec((CI, N), lambda i: (i, 0)),
            pl.BlockSpec((CI, N), lambda i: (i, 0)),
            pl.BlockSpec((D, NB), lambda i: (0, 0)),
            pl.BlockSpec((D, 1), lambda i: (0, 0)),
            pl.BlockSpec((1, D), lambda i: (0, 0)),
            pl.BlockSpec((D, 16), lambda i: (0, 0)),
        ],
        out_specs=pl.BlockSpec((1, 1), lambda i: (0, 0)),
        out_shape=jax.ShapeDtypeStruct((1, 1), jnp.float32),
        scratch_shapes=[pltpu.VMEM((D, N), jnp.float32)],
        compiler_params=pltpu.CompilerParams(
            dimension_semantics=("arbitrary",)),
    )(r2c, spc, wrT, bcol, woT, ett)
    return out.reshape(1)
